# baseline (speedup 1.0000x reference)
import jax
import jax.numpy as jnp
from jax import lax
from jax.experimental import pallas as pl
from jax.experimental.pallas import tpu as pltpu
from jax.experimental.pallas import tpu_sc as plsc

NUM_INPUTS = 512
NUM_COMPUTED = 4096
NUM_OUTPUTS = 128
FAN_IN = 64
SCALE = 4.9
N_UNITS = NUM_INPUTS + 1 + NUM_COMPUTED
CARRY_PAD = 4624
CHUNK = 512
N_CHUNKS = NUM_COMPUTED // CHUNK
GROUPS = CHUNK // 16
OUT_BASE = NUM_INPUTS + 1 + (NUM_COMPUTED - NUM_OUTPUTS)


def _body(x_hbm, w_hbm, idx_hbm, out_hbm, carry, w_v, idx_v, a_ref, st):
    wid = lax.axis_index("s") * 2 + lax.axis_index("c")

    @pl.when(wid == 0)
    def _():
        lane = jnp.arange(16, dtype=jnp.int32)
        ones = jnp.ones((16,), jnp.float32)
        zeros = jnp.zeros((16,), jnp.float32)

        pltpu.sync_copy(x_hbm, carry.at[pl.ds(0, NUM_INPUTS)])

        def init_ones(i, _):
            carry[pl.ds(NUM_INPUTS + 16 * i, 16)] = ones
            return _

        lax.fori_loop(0, (CARRY_PAD - NUM_INPUTS) // 16, init_ones, 0)

        for j in range(16):
            a_ref[j] = zeros

        def group_step(g, pos):
            goff = g * (16 * FAN_IN)
            posv = pos + lane

            nacc = 4
            accs = [jnp.zeros((16,), jnp.float32) for _ in range(nacc)]
            anyint = jnp.zeros((16,), jnp.bool_)
            for k in range(FAN_IN):
                iv = idx_v[pl.ds(goff + 16 * k, 16)]
                wv = w_v[pl.ds(goff + 16 * k, 16)]
                vals = plsc.load_gather(carry, [iv])
                ge = iv >= pos
                internal = jnp.logical_and(ge, iv < posv)
                wm = jnp.where(ge, 0.0, wv)
                selfw = jnp.where(iv == posv, wv, 0.0)
                accs[k % nacc] = accs[k % nacc] + (vals * wm + selfw)
                plsc.addupdate_scatter(a_ref, [iv - pos, lane], wv,
                                       mask=internal)
                anyint = jnp.logical_or(anyint, internal)
            acc_ext = (accs[0] + accs[1]) + (accs[2] + accs[3])

            val = 1.0 / (1.0 + jnp.exp(-SCALE * acc_ext))
            plsc.store_scatter(carry, [posv], val)
            n_int = jnp.sum(anyint.astype(jnp.int32))

            def fix_body(d):
                vcur = plsc.load_gather(carry, [posv])
                corr = jnp.zeros((16,), jnp.float32)
                for j in range(16):
                    vj = plsc.load_gather(carry, [jnp.full((16,), pos + j,
                                                           jnp.int32)])
                    corr = corr + a_ref[j] * vj
                vnew = 1.0 / (1.0 + jnp.exp(-SCALE * (acc_ext + corr)))
                plsc.store_scatter(carry, [posv], vnew)
                return jnp.sum((vnew != vcur).astype(jnp.int32))

            lax.while_loop(lambda d: d > 0, fix_body, n_int)

            @pl.when(n_int > 0)
            def _():
                for j in range(16):
                    a_ref[j] = zeros

            return pos + 16

        def chunk_step(c, pos):
            off = c * (CHUNK * FAN_IN)
            pltpu.sync_copy(w_hbm.at[pl.ds(off, CHUNK * FAN_IN)], w_v)
            pltpu.sync_copy(idx_hbm.at[pl.ds(off, CHUNK * FAN_IN)], idx_v)
            return lax.fori_loop(0, GROUPS, group_step, pos)

        lax.fori_loop(0, N_CHUNKS, chunk_step, NUM_INPUTS + 1)

        for i in range(NUM_OUTPUTS // 16):
            iv = jnp.full((16,), OUT_BASE + 16 * i, jnp.int32) + lane
            st[pl.ds(16 * i, 16)] = plsc.load_gather(carry, [iv])
        pltpu.sync_copy(st, out_hbm)


@jax.jit
def kernel(x, W, input_ids):
    mesh = plsc.VectorSubcoreMesh(core_axis_name="c", subcore_axis_name="s")
    run = pl.kernel(
        _body,
        out_type=jax.ShapeDtypeStruct((NUM_OUTPUTS,), jnp.float32),
        mesh=mesh,
        scratch_types=[
            pltpu.VMEM((CARRY_PAD,), jnp.float32),
            pltpu.VMEM((CHUNK * FAN_IN,), jnp.float32),
            pltpu.VMEM((CHUNK * FAN_IN,), jnp.int32),
            pltpu.VMEM((16, 16), jnp.float32),
            pltpu.VMEM((NUM_OUTPUTS,), jnp.float32),
        ],
        compiler_params=pltpu.CompilerParams(needs_layout_passes=False),
    )
    wT = W.reshape(-1, 16, FAN_IN).transpose(0, 2, 1).reshape(-1)
    idxT = input_ids.reshape(-1, 16, FAN_IN).transpose(0, 2, 1).reshape(-1)
    out = run(x.reshape(-1), wT, idxT)
    return out[None, :]
